# baseline (speedup 1.0000x reference)
import functools

import jax
import jax.numpy as jnp
from jax.experimental import pallas as pl
from jax.experimental.pallas import tpu as pltpu


def _pick_tiles(B, HW):
    tile_b = B
    for tb in (32, 16, 8, 4, 2):
        if B % tb == 0:
            tile_b = tb
            break
    tile_q = HW
    for tq in (7, 8, 4, 2):
        if HW % tq == 0:
            tile_q = tq
            break
    return tile_b, tile_q


def _fused_kernel(x_ref, w1_ref, b1_ref, s_ref, t_ref, w2t_ref, b2_ref,
                  out_ref, acc_ref, *, inv_hw, n_q):
    j = pl.program_id(1)
    part = jnp.sum(x_ref[...], axis=0, dtype=jnp.float32)

    @pl.when(j == 0)
    def _():
        acc_ref[...] = part

    @pl.when(j > 0)
    def _():
        acc_ref[...] += part

    @pl.when(j == n_q - 1)
    def _():
        feat = acc_ref[...] * inv_hw
        h = jnp.dot(feat, w1_ref[...], preferred_element_type=jnp.float32)
        h = h + b1_ref[...]
        h = jnp.maximum(h * s_ref[...] + t_ref[...], 0.0)
        scores_t = jnp.dot(w2t_ref[...], h.T,
                           preferred_element_type=jnp.float32)
        out_ref[...] = scores_t.T + b2_ref[...]


@jax.jit
def _forward(x, w1, b1, bn_scale, bn_shift, w2, b2):
    B, C, H, W = x.shape
    HW = H * W
    D1 = w1.shape[1]
    NC = w2.shape[1]
    xt = jnp.transpose(x, (2, 3, 0, 1)).reshape(HW, B, C)
    w2t = jnp.transpose(w2)
    tile_b, tile_q = _pick_tiles(B, HW)
    n_q = HW // tile_q
    grid = (B // tile_b, n_q)
    body = functools.partial(_fused_kernel, inv_hw=1.0 / float(HW), n_q=n_q)
    return pl.pallas_call(
        body,
        grid=grid,
        in_specs=[
            pl.BlockSpec((tile_q, tile_b, C), lambda i, j: (j, i, 0)),
            pl.BlockSpec((C, D1), lambda i, j: (0, 0)),
            pl.BlockSpec((1, D1), lambda i, j: (0, 0)),
            pl.BlockSpec((1, D1), lambda i, j: (0, 0)),
            pl.BlockSpec((1, D1), lambda i, j: (0, 0)),
            pl.BlockSpec((NC, D1), lambda i, j: (0, 0)),
            pl.BlockSpec((1, NC), lambda i, j: (0, 0)),
        ],
        out_specs=pl.BlockSpec((tile_b, NC), lambda i, j: (i, 0)),
        out_shape=jax.ShapeDtypeStruct((B, NC), jnp.float32),
        scratch_shapes=[pltpu.VMEM((tile_b, C), jnp.float32)],
        compiler_params=pltpu.CompilerParams(
            dimension_semantics=("parallel", "arbitrary")),
        cost_estimate=pl.CostEstimate(
            flops=B * C * HW + 2 * B * C * D1 + 2 * B * D1 * NC,
            transcendentals=0,
            bytes_accessed=(B * C * HW * 4 + C * D1 * 4 + 3 * D1 * 4
                            + D1 * NC * 4 + NC * 4 + B * NC * 4)),
    )(xt, w1, b1, bn_scale, bn_shift, w2t, b2)


def kernel(x, w1, b1, bn_scale, bn_shift, w2, b2):
    return _forward(x, w1, b1, bn_scale, bn_shift, w2, b2)


# speedup vs baseline: 1.4760x; 1.4760x over previous
import functools

import jax
import jax.numpy as jnp
from jax.experimental import pallas as pl
from jax.experimental.pallas import tpu as pltpu


def _pick_b_tile(B):
    for tb in (32, 16, 8, 4, 2):
        if B % tb == 0:
            return tb
    return B


def _fused_kernel(xa_ref, xb_ref, w1a_ref, w1b_ref, b1_ref, s_ref, t_ref,
                  w2t_ref, b2_ref, out_ref, *, inv_hw):
    feat_a = jnp.sum(xa_ref[...], axis=0, dtype=jnp.float32) * inv_hw
    feat_b = jnp.sum(xb_ref[...], axis=0, dtype=jnp.float32) * inv_hw
    h = jnp.dot(feat_a, w1a_ref[...], preferred_element_type=jnp.float32)
    h = h + jnp.dot(feat_b, w1b_ref[...], preferred_element_type=jnp.float32)
    h = h + b1_ref[...]
    h = jnp.maximum(h * s_ref[...] + t_ref[...], 0.0)
    scores_t = jnp.dot(w2t_ref[...], h.T, preferred_element_type=jnp.float32)
    out_ref[...] = scores_t.T + b2_ref[...]


@jax.jit
def _forward(x, w1, b1, bn_scale, bn_shift, w2, b2):
    B, C, H, W = x.shape
    HW = H * W
    D1 = w1.shape[1]
    NC = w2.shape[1]
    xt = jnp.transpose(x, (2, 3, 0, 1)).reshape(HW, B, C)
    w2t = jnp.transpose(w2)
    tile_b = _pick_b_tile(B)
    grid = (B // tile_b,)
    hc = C // 2 if C % 256 == 0 else C
    n_half = 2 if hc != C else 1
    body = functools.partial(_fused_kernel, inv_hw=1.0 / float(HW))
    if n_half == 2:
        return pl.pallas_call(
            body,
            grid=grid,
            in_specs=[
                pl.BlockSpec((HW, tile_b, hc), lambda i: (0, i, 0)),
                pl.BlockSpec((HW, tile_b, hc), lambda i: (0, i, 1)),
                pl.BlockSpec((hc, D1), lambda i: (0, 0)),
                pl.BlockSpec((hc, D1), lambda i: (1, 0)),
                pl.BlockSpec((1, D1), lambda i: (0, 0)),
                pl.BlockSpec((1, D1), lambda i: (0, 0)),
                pl.BlockSpec((1, D1), lambda i: (0, 0)),
                pl.BlockSpec((NC, D1), lambda i: (0, 0)),
                pl.BlockSpec((1, NC), lambda i: (0, 0)),
            ],
            out_specs=pl.BlockSpec((tile_b, NC), lambda i: (i, 0)),
            out_shape=jax.ShapeDtypeStruct((B, NC), jnp.float32),
            compiler_params=pltpu.CompilerParams(
                dimension_semantics=("parallel",)),
            cost_estimate=pl.CostEstimate(
                flops=B * C * HW + 2 * B * C * D1 + 2 * B * D1 * NC,
                transcendentals=0,
                bytes_accessed=(B * C * HW * 4 + C * D1 * 4 + 3 * D1 * 4
                                + D1 * NC * 4 + NC * 4 + B * NC * 4)),
        )(xt, xt, w1, w1, b1, bn_scale, bn_shift, w2t, b2)

    def _one(x_ref, w1_ref, b1_ref, s_ref, t_ref, w2t_ref, b2_ref, out_ref):
        feat = jnp.sum(x_ref[...], axis=0, dtype=jnp.float32) * (1.0 / HW)
        h = jnp.dot(feat, w1_ref[...], preferred_element_type=jnp.float32)
        h = h + b1_ref[...]
        h = jnp.maximum(h * s_ref[...] + t_ref[...], 0.0)
        scores_t = jnp.dot(w2t_ref[...], h.T,
                           preferred_element_type=jnp.float32)
        out_ref[...] = scores_t.T + b2_ref[...]

    return pl.pallas_call(
        _one,
        grid=grid,
        in_specs=[
            pl.BlockSpec((HW, tile_b, C), lambda i: (0, i, 0)),
            pl.BlockSpec((C, D1), lambda i: (0, 0)),
            pl.BlockSpec((1, D1), lambda i: (0, 0)),
            pl.BlockSpec((1, D1), lambda i: (0, 0)),
            pl.BlockSpec((1, D1), lambda i: (0, 0)),
            pl.BlockSpec((NC, D1), lambda i: (0, 0)),
            pl.BlockSpec((1, NC), lambda i: (0, 0)),
        ],
        out_specs=pl.BlockSpec((tile_b, NC), lambda i: (i, 0)),
        out_shape=jax.ShapeDtypeStruct((B, NC), jnp.float32),
        compiler_params=pltpu.CompilerParams(
            dimension_semantics=("parallel",)),
        cost_estimate=pl.CostEstimate(
            flops=B * C * HW + 2 * B * C * D1 + 2 * B * D1 * NC,
            transcendentals=0,
            bytes_accessed=(B * C * HW * 4 + C * D1 * 4 + 3 * D1 * 4
                            + D1 * NC * 4 + NC * 4 + B * NC * 4)),
    )(xt, w1, b1, bn_scale, bn_shift, w2t, b2)


def kernel(x, w1, b1, bn_scale, bn_shift, w2, b2):
    return _forward(x, w1, b1, bn_scale, bn_shift, w2, b2)


# speedup vs baseline: 1.4889x; 1.0087x over previous
import functools

import jax
import jax.numpy as jnp
from jax.experimental import pallas as pl
from jax.experimental.pallas import tpu as pltpu


def _pick_b_tile(B):
    for tb in (32, 16, 8, 4, 2):
        if B % tb == 0:
            return tb
    return B


def _fused_kernel(*refs, inv_hw, n_s):
    x_refs = refs[:n_s]
    w1_refs = refs[n_s:2 * n_s]
    b1_ref, s_ref, t_ref, w2t_ref, b2_ref, out_ref = refs[2 * n_s:]
    h = None
    for x_ref, w1_ref in zip(x_refs, w1_refs):
        feat = jnp.sum(x_ref[...], axis=0, dtype=jnp.float32) * inv_hw
        part = jnp.dot(feat, w1_ref[...], preferred_element_type=jnp.float32)
        h = part if h is None else h + part
    h = h + b1_ref[...]
    h = jnp.maximum(h * s_ref[...] + t_ref[...], 0.0)
    scores_t = jnp.dot(w2t_ref[...], h.T, preferred_element_type=jnp.float32)
    out_ref[...] = scores_t.T + b2_ref[...]


@jax.jit
def _forward(x, w1, b1, bn_scale, bn_shift, w2, b2):
    B, C, H, W = x.shape
    HW = H * W
    D1 = w1.shape[1]
    NC = w2.shape[1]
    xt = jnp.transpose(x, (2, 3, 0, 1)).reshape(HW, B, C)
    w2t = jnp.transpose(w2)
    tile_b = _pick_b_tile(B)
    grid = (B // tile_b,)
    n_s = 4 if C % 512 == 0 else (2 if C % 256 == 0 else 1)
    sc = C // n_s
    body = functools.partial(_fused_kernel, inv_hw=1.0 / float(HW), n_s=n_s)

    def _xmap(k):
        return lambda i: (0, i, k)

    def _wmap(k):
        return lambda i: (k, 0)

    in_specs = (
        [pl.BlockSpec((HW, tile_b, sc), _xmap(k)) for k in range(n_s)]
        + [pl.BlockSpec((sc, D1), _wmap(k)) for k in range(n_s)]
        + [
            pl.BlockSpec((1, D1), lambda i: (0, 0)),
            pl.BlockSpec((1, D1), lambda i: (0, 0)),
            pl.BlockSpec((1, D1), lambda i: (0, 0)),
            pl.BlockSpec((NC, D1), lambda i: (0, 0)),
            pl.BlockSpec((1, NC), lambda i: (0, 0)),
        ])
    return pl.pallas_call(
        body,
        grid=grid,
        in_specs=in_specs,
        out_specs=pl.BlockSpec((tile_b, NC), lambda i: (i, 0)),
        out_shape=jax.ShapeDtypeStruct((B, NC), jnp.float32),
        compiler_params=pltpu.CompilerParams(
            dimension_semantics=("parallel",)),
        cost_estimate=pl.CostEstimate(
            flops=B * C * HW + 2 * B * C * D1 + 2 * B * D1 * NC,
            transcendentals=0,
            bytes_accessed=(B * C * HW * 4 + C * D1 * 4 + 3 * D1 * 4
                            + D1 * NC * 4 + NC * 4 + B * NC * 4)),
    )(*([xt] * n_s), *([w1] * n_s), b1, bn_scale, bn_shift, w2t, b2)


def kernel(x, w1, b1, bn_scale, bn_shift, w2, b2):
    return _forward(x, w1, b1, bn_scale, bn_shift, w2, b2)


# speedup vs baseline: 1.4892x; 1.0002x over previous
import functools

import jax
import jax.numpy as jnp
from jax.experimental import pallas as pl
from jax.experimental.pallas import tpu as pltpu


def _pick_b_tile(B):
    for tb in (32, 16, 8, 4, 2):
        if B % tb == 0:
            return tb
    return B


def _fused_kernel(*refs, inv_hw, n_s):
    x_refs = refs[:n_s]
    w1_refs = refs[n_s:2 * n_s]
    b1_ref, s_ref, t_ref, w2t_ref, b2_ref, out_ref = refs[2 * n_s:]
    h = None
    for x_ref, w1_ref in zip(x_refs, w1_refs):
        feat = jnp.sum(x_ref[...], axis=0, dtype=jnp.float32) * inv_hw
        part = jnp.dot(feat, w1_ref[...], preferred_element_type=jnp.float32)
        h = part if h is None else h + part
    h = h + b1_ref[...]
    h = jnp.maximum(h * s_ref[...] + t_ref[...], 0.0)
    scores_t = jnp.dot(w2t_ref[...], h.T, preferred_element_type=jnp.float32)
    out_ref[...] = scores_t.T + b2_ref[...]


@jax.jit
def _forward(x, w1, b1, bn_scale, bn_shift, w2, b2):
    B, C, H, W = x.shape
    HW = H * W
    D1 = w1.shape[1]
    NC = w2.shape[1]
    xt = jnp.transpose(x, (2, 3, 0, 1)).reshape(HW, B, C)
    w2t = jnp.transpose(w2)
    tile_b = _pick_b_tile(B)
    grid = (B // tile_b,)
    n_s = 8 if C % 1024 == 0 else (4 if C % 512 == 0 else (2 if C % 256 == 0 else 1))
    sc = C // n_s
    body = functools.partial(_fused_kernel, inv_hw=1.0 / float(HW), n_s=n_s)

    def _xmap(k):
        return lambda i: (0, i, k)

    def _wmap(k):
        return lambda i: (k, 0)

    in_specs = (
        [pl.BlockSpec((HW, tile_b, sc), _xmap(k)) for k in range(n_s)]
        + [pl.BlockSpec((sc, D1), _wmap(k)) for k in range(n_s)]
        + [
            pl.BlockSpec((1, D1), lambda i: (0, 0)),
            pl.BlockSpec((1, D1), lambda i: (0, 0)),
            pl.BlockSpec((1, D1), lambda i: (0, 0)),
            pl.BlockSpec((NC, D1), lambda i: (0, 0)),
            pl.BlockSpec((1, NC), lambda i: (0, 0)),
        ])
    return pl.pallas_call(
        body,
        grid=grid,
        in_specs=in_specs,
        out_specs=pl.BlockSpec((tile_b, NC), lambda i: (i, 0)),
        out_shape=jax.ShapeDtypeStruct((B, NC), jnp.float32),
        compiler_params=pltpu.CompilerParams(
            dimension_semantics=("parallel",)),
        cost_estimate=pl.CostEstimate(
            flops=B * C * HW + 2 * B * C * D1 + 2 * B * D1 * NC,
            transcendentals=0,
            bytes_accessed=(B * C * HW * 4 + C * D1 * 4 + 3 * D1 * 4
                            + D1 * NC * 4 + NC * 4 + B * NC * 4)),
    )(*([xt] * n_s), *([w1] * n_s), b1, bn_scale, bn_shift, w2t, b2)


def kernel(x, w1, b1, bn_scale, bn_shift, w2, b2):
    return _forward(x, w1, b1, bn_scale, bn_shift, w2, b2)


# speedup vs baseline: 1.4916x; 1.0016x over previous
import functools

import jax
import jax.numpy as jnp
from jax.experimental import pallas as pl
from jax.experimental.pallas import tpu as pltpu


def _pick_b_tile(B):
    for tb in (32, 16, 8, 4, 2):
        if B % tb == 0:
            return tb
    return B


def _fused_kernel(*refs, inv_hw, n_s):
    x_refs = refs[:n_s]
    w1_refs = refs[n_s:2 * n_s]
    b1_ref, s_ref, t_ref, w2t_ref, b2_ref, out_ref = refs[2 * n_s:]
    h = None
    for x_ref, w1_ref in zip(x_refs, w1_refs):
        feat = jnp.sum(x_ref[...], axis=0, dtype=jnp.float32) * inv_hw
        part = jnp.dot(feat, w1_ref[...], preferred_element_type=jnp.float32)
        h = part if h is None else h + part
    h = h + b1_ref[...]
    h = jnp.maximum(h * s_ref[...] + t_ref[...], 0.0)
    scores_t = jnp.dot(w2t_ref[...], h.T, preferred_element_type=jnp.float32)
    out_ref[...] = scores_t.T + b2_ref[...]


@jax.jit
def _forward(x, w1, b1, bn_scale, bn_shift, w2, b2):
    B, C, H, W = x.shape
    HW = H * W
    D1 = w1.shape[1]
    NC = w2.shape[1]
    xt = jnp.transpose(x, (2, 3, 0, 1)).reshape(HW, B, C)
    w2t = jnp.transpose(w2)
    tile_b = _pick_b_tile(B)
    grid = (B // tile_b,)
    n_s = 4 if C % 512 == 0 else (2 if C % 256 == 0 else 1)
    sc = C // n_s
    body = functools.partial(_fused_kernel, inv_hw=1.0 / float(HW), n_s=n_s)

    def _xmap(k):
        return lambda i: (0, i, k)

    def _wmap(k):
        return lambda i: (k, 0)

    in_specs = (
        [pl.BlockSpec((HW, tile_b, sc), _xmap(k)) for k in range(n_s)]
        + [pl.BlockSpec((sc, D1), _wmap(k)) for k in range(n_s)]
        + [
            pl.BlockSpec((1, D1), lambda i: (0, 0)),
            pl.BlockSpec((1, D1), lambda i: (0, 0)),
            pl.BlockSpec((1, D1), lambda i: (0, 0)),
            pl.BlockSpec((NC, D1), lambda i: (0, 0)),
            pl.BlockSpec((1, NC), lambda i: (0, 0)),
        ])
    return pl.pallas_call(
        body,
        grid=grid,
        in_specs=in_specs,
        out_specs=pl.BlockSpec((tile_b, NC), lambda i: (i, 0)),
        out_shape=jax.ShapeDtypeStruct((B, NC), jnp.float32),
        compiler_params=pltpu.CompilerParams(
            dimension_semantics=("parallel",)),
        cost_estimate=pl.CostEstimate(
            flops=B * C * HW + 2 * B * C * D1 + 2 * B * D1 * NC,
            transcendentals=0,
            bytes_accessed=(B * C * HW * 4 + C * D1 * 4 + 3 * D1 * 4
                            + D1 * NC * 4 + NC * 4 + B * NC * 4)),
    )(*([xt] * n_s), *([w1] * n_s), b1, bn_scale, bn_shift, w2t, b2)


def kernel(x, w1, b1, bn_scale, bn_shift, w2, b2):
    return _forward(x, w1, b1, bn_scale, bn_shift, w2, b2)
